# baseline (speedup 1.0000x reference)
import jax
import jax.numpy as jnp
from jax.experimental import pallas as pl

_BATCH = 16384
_TOKENS = 100
_LANES = 128
_ROWS = (_BATCH * _TOKENS) // _LANES
_BLOCK_ROWS = 1600


def _zeros_mask_kernel(o_ref):
    o_ref[...] = jnp.zeros_like(o_ref)


def kernel(inputs):
    mask2d = pl.pallas_call(
        _zeros_mask_kernel,
        out_shape=jax.ShapeDtypeStruct((_ROWS, _LANES), inputs.dtype),
        grid=(_ROWS // _BLOCK_ROWS,),
        out_specs=pl.BlockSpec((_BLOCK_ROWS, _LANES), lambda i: (i, 0)),
    )()
    mask = mask2d.reshape(_BATCH, _TOKENS, 1)
    return (inputs, mask)
